# baseline (speedup 1.0000x reference)
import jax
import jax.numpy as jnp
from jax.experimental import pallas as pl
from jax.experimental.pallas import tpu as pltpu
from jax.experimental.pallas import tpu_sc as plsc


def kernel(token_ids, weights):
    B, S = token_ids.shape
    V, D = weights.shape

    W = 512
    K = 2048
    assert B % W == 0 and K % 4 == 0

    idx = token_ids.T.astype(jnp.int32)
    wT = weights.T

    mesh = plsc.VectorSubcoreMesh(
        core_axis_name="core", subcore_axis_name="subcore"
    )

    @jax.jit
    def run(wT, idx):
        def pack_body(in_ref, out_ref):
            x = in_ref[...]
            y = jnp.transpose(x)
            y3 = y.reshape(K // 4, 4, D)
            out_ref[...] = jnp.concatenate(
                [y3[:, j, :] for j in range(4)], axis=1
            )

        packed = jax.lax.optimization_barrier(
            jnp.transpose(wT).reshape(V // 4, 4 * D)
        )
        table = packed.reshape(V, D)

        @pl.kernel(
            out_type=jax.ShapeDtypeStruct((S, B, D), wT.dtype),
            mesh=mesh,
            compiler_params=pltpu.CompilerParams(use_tc_tiling_on_sc=False),
        )
        def gather_kernel(w_hbm, i_hbm, o_hbm):
            def body(i_vmem, o_vmem):
                pltpu.sync_copy(w_hbm.at[i_vmem.at[0]], o_vmem.at[0])

            pltpu.emit_pipeline(
                body,
                grid=(S, B // W),
                in_specs=[
                    pl.BlockSpec((1, W), index_map=lambda s, i: (s, i))
                ],
                out_specs=[
                    pl.BlockSpec((1, W, D), index_map=lambda s, i: (s, i, 0))
                ],
                core_axis_name=("core", "subcore"),
                dimension_semantics=(pltpu.PARALLEL, pltpu.PARALLEL),
            )(i_hbm, o_hbm)

        return gather_kernel(table, idx)

    out = run(wT, idx)
    return jnp.transpose(out, (1, 0, 2))


# speedup vs baseline: 1.0128x; 1.0128x over previous
import jax
import jax.numpy as jnp
from jax.experimental import pallas as pl
from jax.experimental.pallas import tpu as pltpu
from jax.experimental.pallas import tpu_sc as plsc


def kernel(token_ids, weights):
    B, S = token_ids.shape
    V, D = weights.shape

    W = 512
    K = 2048
    assert B % W == 0 and K % 4 == 0

    idx = token_ids.T.astype(jnp.int32)
    wT = weights.T

    mesh = plsc.VectorSubcoreMesh(
        core_axis_name="core", subcore_axis_name="subcore"
    )

    @jax.jit
    def run(wT, idx):
        def pack_body(in_ref, out_ref):
            x = in_ref[...]
            y = jnp.transpose(x)
            y3 = y.reshape(K // 4, 4, D)
            out_ref[...] = jnp.concatenate(
                [y3[:, j, :] for j in range(4)], axis=1
            )

        packed = pl.pallas_call(
            pack_body,
            grid=((V + K - 1) // K,),
            in_specs=[pl.BlockSpec((D, K), lambda i: (0, i))],
            out_specs=pl.BlockSpec((K // 4, 4 * D), lambda i: (i, 0)),
            out_shape=jax.ShapeDtypeStruct((V // 4, 4 * D), wT.dtype),
        )(wT)
        table = packed.reshape(V, D)

        @pl.kernel(
            out_type=jax.ShapeDtypeStruct((S, B, D), wT.dtype),
            mesh=mesh,
            compiler_params=pltpu.CompilerParams(use_tc_tiling_on_sc=False),
        )
        def gather_kernel(w_hbm, i_hbm, o_hbm):
            def body(i_vmem, o_vmem):
                pltpu.sync_copy(w_hbm.at[i_vmem.at[0]], o_vmem.at[0])

            pltpu.emit_pipeline(
                body,
                grid=(S, B // W),
                in_specs=[
                    pl.BlockSpec((1, W), index_map=lambda s, i: (s, i))
                ],
                out_specs=[
                    pl.BlockSpec((1, W, D), index_map=lambda s, i: (s, i, 0))
                ],
                core_axis_name=("core", "subcore"),
                dimension_semantics=(pltpu.PARALLEL, pltpu.PARALLEL),
            )(i_hbm, o_hbm)

        return gather_kernel(table, idx)

    out = run(wT, idx)
    return jnp.transpose(out, (1, 0, 2))
